# baseline (speedup 1.0000x reference)
import math

import jax
import jax.numpy as jnp
import numpy as np
from jax.experimental import pallas as pl
from jax.experimental.pallas import tpu as pltpu

_B, _S, _D, _H = 1, 2048, 1024, 16
_QL, _KVL, _NOPE, _ROPE, _VH = 1536, 512, 128, 64, 128
_QH = _NOPE + _ROPE
_INTER = 512
_E, _NSH, _NG, _TKG, _TOPK = 16, 2, 4, 2, 4
_GS = _E // _NG


def _ln_body(x, w, b, eps=1e-5):
    m = jnp.mean(x, -1, keepdims=True)
    v = jnp.mean((x - m) ** 2, -1, keepdims=True)
    return (x - m) / jnp.sqrt(v + eps) * w + b


def _dot_t(a, bt):
    return jax.lax.dot_general(a.astype(jnp.bfloat16), bt.astype(jnp.bfloat16),
                               (((1,), (1,)), ((), ())),
                               preferred_element_type=jnp.float32)


def _ln_mm1_kernel(x_ref, lnw_ref, lnb_ref, wt_ref, o_ref):
    h = _ln_body(x_ref[...], lnw_ref[0], lnb_ref[0])
    o_ref[...] = _dot_t(h, wt_ref[...])


def _ln_mm1(x, lnw, lnb, wt):
    m, k = x.shape
    n = wt.shape[0]
    return pl.pallas_call(
        _ln_mm1_kernel,
        out_shape=jax.ShapeDtypeStruct((m, n), jnp.float32),
    )(x, lnw.reshape(1, k), lnb.reshape(1, k), wt)


def _ln_mm_kv_kernel(x_ref, lnw_ref, lnb_ref, wt_ref, ckv_ref, kpe_ref):
    h = _ln_body(x_ref[...], lnw_ref[0], lnb_ref[0])
    r = _dot_t(h, wt_ref[...])
    ckv_ref[...] = r[:, :_KVL]
    kpe_ref[...] = r[:, _KVL:]


def _ln_mm_kv(x, lnw, lnb, wt):
    m, k = x.shape
    return pl.pallas_call(
        _ln_mm_kv_kernel,
        out_shape=[
            jax.ShapeDtypeStruct((m, _KVL), jnp.float32),
            jax.ShapeDtypeStruct((m, _ROPE), jnp.float32),
        ],
    )(x, lnw.reshape(1, k), lnb.reshape(1, k), wt)


def _ln_mm_heads_kernel(x_ref, lnw_ref, lnb_ref, wt_ref, oa_ref, ob_ref,
                        scratch):
    h = pl.program_id(0)

    @pl.when(h == 0)
    def _():
        scratch[...] = _ln_body(x_ref[...], lnw_ref[0], lnb_ref[0])

    r = _dot_t(scratch[...], wt_ref[0])
    na = oa_ref.shape[2]
    oa_ref[0] = r[:, :na]
    ob_ref[0] = r[:, na:]


def _ln_mm_heads(x, lnw, lnb, wt3, na, nb):
    m, k = x.shape
    return pl.pallas_call(
        _ln_mm_heads_kernel,
        grid=(_H,),
        in_specs=[
            pl.BlockSpec((m, k), lambda h: (0, 0)),
            pl.BlockSpec((1, k), lambda h: (0, 0)),
            pl.BlockSpec((1, k), lambda h: (0, 0)),
            pl.BlockSpec((1, na + nb, k), lambda h: (h, 0, 0)),
        ],
        out_specs=[
            pl.BlockSpec((1, m, na), lambda h: (h, 0, 0)),
            pl.BlockSpec((1, m, nb), lambda h: (h, 0, 0)),
        ],
        out_shape=[
            jax.ShapeDtypeStruct((_H, m, na), jnp.float32),
            jax.ShapeDtypeStruct((_H, m, nb), jnp.float32),
        ],
        scratch_shapes=[pltpu.VMEM((m, k), jnp.float32)],
    )(x, lnw.reshape(1, k), lnb.reshape(1, k), wt3)


def _rope(x, cos, sin):
    a = jnp.roll(x, -1, axis=1)
    b = jnp.roll(x, 1, axis=1)
    lane = jax.lax.broadcasted_iota(jnp.int32, x.shape, 1)
    rot = jnp.where(lane % 2 == 0, -a, b)
    return x * cos + rot * sin


_BQ = 256
_BK = 256


def _attn_kernel(qn_ref, qp_ref, kn_ref, v_ref, kpe_ref, cosq_ref, sinq_ref,
                 cosk_ref, sink_ref, o_ref):
    qb = pl.program_id(1)
    qpe = _rope(qp_ref[0], cosq_ref[...], sinq_ref[...])
    qn_l = qn_ref[0]
    scale = 1.0 / math.sqrt(_QH)
    rowg = qb * _BQ + jax.lax.broadcasted_iota(jnp.int32, (_BQ, _BK), 0)

    def body(kb, carry):
        m, l, acc = carry
        base = kb * _BK
        kn_b = kn_ref[0, pl.ds(base, _BK), :]
        kpe_b = _rope(kpe_ref[pl.ds(base, _BK), :],
                      cosk_ref[pl.ds(base, _BK), :],
                      sink_ref[pl.ds(base, _BK), :])
        v_b = v_ref[0, pl.ds(base, _BK), :]
        s = (_dot_t(qn_l, kn_b) + _dot_t(qpe, kpe_b)) * scale
        colg = base + jax.lax.broadcasted_iota(jnp.int32, (_BQ, _BK), 1)
        s = jnp.where(colg > rowg, jnp.float32(-1e9), s)
        m_new = jnp.maximum(m, jnp.max(s, axis=1, keepdims=True))
        alpha = jnp.exp(m - m_new)
        p = jnp.exp(s - m_new)
        l_new = l * alpha + jnp.sum(p, axis=1, keepdims=True)
        pv = jax.lax.dot_general(p.astype(jnp.bfloat16),
                                 v_b.astype(jnp.bfloat16),
                                 (((1,), (0,)), ((), ())),
                                 preferred_element_type=jnp.float32)
        return m_new, l_new, acc * alpha + pv

    nkb = (qb * _BQ + _BQ + _BK - 1) // _BK
    init = (jnp.full((_BQ, 1), -1e30, jnp.float32),
            jnp.zeros((_BQ, 1), jnp.float32),
            jnp.zeros((_BQ, _VH), jnp.float32))
    m, l, acc = jax.lax.fori_loop(0, nkb, body, init)
    o_ref[0] = acc / l


def _attention(qn, qp, kn, v, kpe, cos2, sin2):
    return pl.pallas_call(
        _attn_kernel,
        grid=(_H, _S // _BQ),
        in_specs=[
            pl.BlockSpec((1, _BQ, _NOPE), lambda h, qb: (h, qb, 0)),
            pl.BlockSpec((1, _BQ, _ROPE), lambda h, qb: (h, qb, 0)),
            pl.BlockSpec((1, _S, _NOPE), lambda h, qb: (h, 0, 0)),
            pl.BlockSpec((1, _S, _VH), lambda h, qb: (h, 0, 0)),
            pl.BlockSpec((_S, _ROPE), lambda h, qb: (0, 0)),
            pl.BlockSpec((_BQ, _ROPE), lambda h, qb: (qb, 0)),
            pl.BlockSpec((_BQ, _ROPE), lambda h, qb: (qb, 0)),
            pl.BlockSpec((_S, _ROPE), lambda h, qb: (0, 0)),
            pl.BlockSpec((_S, _ROPE), lambda h, qb: (0, 0)),
        ],
        out_specs=pl.BlockSpec((1, _BQ, _VH), lambda h, qb: (h, qb, 0)),
        out_shape=jax.ShapeDtypeStruct((_H, _S, _VH), jnp.float32),
    )(qn, qp, kn, v, kpe, cos2, sin2, cos2, sin2)


def _oproj_kernel(ao_ref, ow_ref, x_ref, ln2w_ref, ln2b_ref, h1_ref, xf_ref):
    h = pl.program_id(0)
    part = _dot_t(ao_ref[0], ow_ref[...])

    @pl.when(h == 0)
    def _():
        h1_ref[...] = x_ref[...] + part

    @pl.when(h > 0)
    def _():
        h1_ref[...] += part

    @pl.when(h == _H - 1)
    def _():
        xf_ref[...] = _ln_body(h1_ref[...], ln2w_ref[0], ln2b_ref[0])


def _oproj(ao, o_w, x, ln2w, ln2b):
    return pl.pallas_call(
        _oproj_kernel,
        grid=(_H,),
        in_specs=[
            pl.BlockSpec((1, _S, _VH), lambda h: (h, 0, 0)),
            pl.BlockSpec((_D, _VH), lambda h: (0, h)),
            pl.BlockSpec((_S, _D), lambda h: (0, 0)),
            pl.BlockSpec((1, _D), lambda h: (0, 0)),
            pl.BlockSpec((1, _D), lambda h: (0, 0)),
        ],
        out_specs=[
            pl.BlockSpec((_S, _D), lambda h: (0, 0)),
            pl.BlockSpec((_S, _D), lambda h: (0, 0)),
        ],
        out_shape=[
            jax.ShapeDtypeStruct((_S, _D), jnp.float32),
            jax.ShapeDtypeStruct((_S, _D), jnp.float32),
        ],
    )(ao, o_w, x, ln2w.reshape(1, _D), ln2b.reshape(1, _D))


def _route_kernel(xf_ref, gw_ref, comb_ref):
    bm = xf_ref.shape[0]
    l = jax.lax.dot_general(xf_ref[...], gw_ref[...], (((1,), (1,)), ((), ())),
                            preferred_element_type=jnp.float32)
    ivec = jax.lax.broadcasted_iota(jnp.int32, (bm, _E), 1)
    r = jnp.zeros((bm, _E), jnp.float32)
    for j in range(_E):
        lj = jax.lax.slice_in_dim(l, j, j + 1, axis=1)
        beats = (lj > l) | ((lj == l) & (j < ivec))
        sg = (ivec // _GS) == (j // _GS)
        r = r + jnp.where(beats & sg, 1.0, 0.0)
    cand = r < _TKG
    pos = (ivec // _GS).astype(jnp.float32) * _TKG + r
    rr = jnp.zeros((bm, _E), jnp.float32)
    for j in range(_E):
        lj = jax.lax.slice_in_dim(l, j, j + 1, axis=1)
        pj = jax.lax.slice_in_dim(pos, j, j + 1, axis=1)
        cj = jax.lax.slice_in_dim(cand, j, j + 1, axis=1)
        beats2 = cj & ((lj > l) | ((lj == l) & (pj < pos)))
        rr = rr + jnp.where(beats2, 1.0, 0.0)
    sel = cand & (rr < _TOPK)
    w = jnp.where(sel, l, jnp.float32(0.0))
    comb_ref[...] = w / (jnp.sum(w, axis=1, keepdims=True) + 1e-20)


def _route(xf, gate_w, bm=512):
    return pl.pallas_call(
        _route_kernel,
        grid=(_S // bm,),
        in_specs=[
            pl.BlockSpec((bm, _D), lambda i: (i, 0)),
            pl.BlockSpec((_E, _D), lambda i: (0, 0)),
        ],
        out_specs=pl.BlockSpec((bm, _E), lambda i: (i, 0)),
        out_shape=jax.ShapeDtypeStruct((_S, _E), jnp.float32),
    )(xf, gate_w)


def _shared_kernel(xf_ref, h1_ref, gu_ref, dn_ref, o_ref):
    e = pl.program_id(0)
    h = _dot_t(xf_ref[...], gu_ref[0])
    g = h[:, :_INTER]
    u = h[:, _INTER:]
    act = (g / (1.0 + jnp.exp(-g))) * u
    y = _dot_t(act, dn_ref[0])

    @pl.when(e == 0)
    def _():
        o_ref[...] = h1_ref[...] + y

    @pl.when(e > 0)
    def _():
        o_ref[...] += y


def _shared(xf, h1, sh_gu, sh_dn):
    return pl.pallas_call(
        _shared_kernel,
        grid=(_NSH,),
        in_specs=[
            pl.BlockSpec((_S, _D), lambda e: (0, 0)),
            pl.BlockSpec((_S, _D), lambda e: (0, 0)),
            pl.BlockSpec((1, 2 * _INTER, _D), lambda e: (e, 0, 0)),
            pl.BlockSpec((1, _D, _INTER), lambda e: (e, 0, 0)),
        ],
        out_specs=pl.BlockSpec((_S, _D), lambda e: (0, 0)),
        out_shape=jax.ShapeDtypeStruct((_S, _D), jnp.float32),
    )(xf, h1, sh_gu, sh_dn)


def _routed_kernel(xf_ref, t1_ref, comb_ref, gu_ref, dn_ref, o_ref):
    e = pl.program_id(0)
    h = _dot_t(xf_ref[...], gu_ref[0])
    g = h[:, :_INTER]
    u = h[:, _INTER:]
    act = (g / (1.0 + jnp.exp(-g))) * u
    y = _dot_t(act, dn_ref[0])
    lane = jax.lax.broadcasted_iota(jnp.int32, (_S, _E), 1)
    c = jnp.sum(jnp.where(lane == e, comb_ref[...], 0.0), axis=1,
                keepdims=True)

    @pl.when(e == 0)
    def _():
        o_ref[...] = t1_ref[...] + c * y

    @pl.when(e > 0)
    def _():
        o_ref[...] += c * y


def _routed(xf, t1, comb, exp_gu, exp_dn):
    return pl.pallas_call(
        _routed_kernel,
        grid=(_E,),
        in_specs=[
            pl.BlockSpec((_S, _D), lambda e: (0, 0)),
            pl.BlockSpec((_S, _D), lambda e: (0, 0)),
            pl.BlockSpec((_S, _E), lambda e: (0, 0)),
            pl.BlockSpec((1, 2 * _INTER, _D), lambda e: (e, 0, 0)),
            pl.BlockSpec((1, _D, _INTER), lambda e: (e, 0, 0)),
        ],
        out_specs=pl.BlockSpec((_S, _D), lambda e: (0, 0)),
        out_shape=jax.ShapeDtypeStruct((_S, _D), jnp.float32),
    )(xf, t1, comb, exp_gu, exp_dn)


def _rope_tables():
    inv = 1.0 / (10000.0 ** (np.arange(0, _ROPE, 2, dtype=np.float64) / _ROPE))
    t = np.arange(_S, dtype=np.float64)
    f = np.outer(t, inv)
    cos2 = np.repeat(np.cos(f), 2, axis=1)
    sin2 = np.repeat(np.sin(f), 2, axis=1)
    return jnp.asarray(cos2, jnp.float32), jnp.asarray(sin2, jnp.float32)


def kernel(x, ln1_w, ln1_b, ln2_w, ln2_b, qkv_a_w, qa_ln_w, qa_ln_b, q_b_w,
           kv_a_w, kv_ln_w, kv_ln_b, kv_b_w, o_w, gate_w, exp_gu, exp_dn,
           sh_gu, sh_dn):
    x2 = x.reshape(_S, _D)
    cos2, sin2 = _rope_tables()

    q_lat = _ln_mm1(x2, ln1_w, ln1_b, qkv_a_w[:_QL])
    c_kv, k_pe = _ln_mm_kv(x2, ln1_w, ln1_b, kv_a_w)

    qn, qp = _ln_mm_heads(q_lat, qa_ln_w, qa_ln_b,
                          q_b_w.reshape(_H, _QH, _QL), _NOPE, _ROPE)
    kn, v3 = _ln_mm_heads(c_kv, kv_ln_w, kv_ln_b,
                          kv_b_w.reshape(_H, _NOPE + _VH, _KVL), _NOPE, _VH)

    ao = _attention(qn, qp, kn, v3, k_pe, cos2, sin2)
    h1, xf = _oproj(ao, o_w, x2, ln2_w, ln2_b)

    comb = _route(xf, gate_w)
    t1 = _shared(xf, h1, sh_gu, sh_dn)
    out = _routed(xf, t1, comb, exp_gu, exp_dn)

    return out.reshape(_B, _S, _D)


# speedup vs baseline: 1.4052x; 1.4052x over previous
import math

import jax
import jax.numpy as jnp
import numpy as np
from jax.experimental import pallas as pl
from jax.experimental.pallas import tpu as pltpu

_B, _S, _D, _H = 1, 2048, 1024, 16
_QL, _KVL, _NOPE, _ROPE, _VH = 1536, 512, 128, 64, 128
_QH = _NOPE + _ROPE
_INTER = 512
_E, _NSH, _NG, _TKG, _TOPK = 16, 2, 4, 2, 4
_GS = _E // _NG


def _ln_body(x, w, b, eps=1e-5):
    m = jnp.mean(x, -1, keepdims=True)
    v = jnp.mean((x - m) ** 2, -1, keepdims=True)
    return (x - m) / jnp.sqrt(v + eps) * w + b


def _dot_t(a, bt):
    return jax.lax.dot_general(a.astype(jnp.bfloat16), bt.astype(jnp.bfloat16),
                               (((1,), (1,)), ((), ())),
                               preferred_element_type=jnp.float32)


def _ln_mm1_kernel(x_ref, lnw_ref, lnb_ref, wt_ref, o_ref):
    h = _ln_body(x_ref[...], lnw_ref[0], lnb_ref[0])
    o_ref[...] = _dot_t(h, wt_ref[...])


def _ln_mm1(x, lnw, lnb, wt):
    m, k = x.shape
    n = wt.shape[0]
    return pl.pallas_call(
        _ln_mm1_kernel,
        out_shape=jax.ShapeDtypeStruct((m, n), jnp.float32),
    )(x, lnw.reshape(1, k), lnb.reshape(1, k), wt)


def _ln_mm_kv_kernel(x_ref, lnw_ref, lnb_ref, wt_ref, ckv_ref, kpe_ref):
    h = _ln_body(x_ref[...], lnw_ref[0], lnb_ref[0])
    r = _dot_t(h, wt_ref[...])
    ckv_ref[...] = r[:, :_KVL]
    kpe_ref[...] = r[:, _KVL:]


def _ln_mm_kv(x, lnw, lnb, wt):
    m, k = x.shape
    return pl.pallas_call(
        _ln_mm_kv_kernel,
        out_shape=[
            jax.ShapeDtypeStruct((m, _KVL), jnp.float32),
            jax.ShapeDtypeStruct((m, _ROPE), jnp.float32),
        ],
    )(x, lnw.reshape(1, k), lnb.reshape(1, k), wt)


def _ln_mm_heads_kernel(x_ref, lnw_ref, lnb_ref, wt_ref, oa_ref, ob_ref,
                        scratch):
    h = pl.program_id(0)

    @pl.when(h == 0)
    def _():
        scratch[...] = _ln_body(x_ref[...], lnw_ref[0], lnb_ref[0])

    r = _dot_t(scratch[...], wt_ref[0])
    na = oa_ref.shape[2]
    oa_ref[0] = r[:, :na]
    ob_ref[0] = r[:, na:]


def _ln_mm_heads(x, lnw, lnb, wt3, na, nb):
    m, k = x.shape
    return pl.pallas_call(
        _ln_mm_heads_kernel,
        grid=(_H,),
        in_specs=[
            pl.BlockSpec((m, k), lambda h: (0, 0)),
            pl.BlockSpec((1, k), lambda h: (0, 0)),
            pl.BlockSpec((1, k), lambda h: (0, 0)),
            pl.BlockSpec((1, na + nb, k), lambda h: (h, 0, 0)),
        ],
        out_specs=[
            pl.BlockSpec((1, m, na), lambda h: (h, 0, 0)),
            pl.BlockSpec((1, m, nb), lambda h: (h, 0, 0)),
        ],
        out_shape=[
            jax.ShapeDtypeStruct((_H, m, na), jnp.float32),
            jax.ShapeDtypeStruct((_H, m, nb), jnp.float32),
        ],
        scratch_shapes=[pltpu.VMEM((m, k), jnp.float32)],
    )(x, lnw.reshape(1, k), lnb.reshape(1, k), wt3)


def _rope(x, cos, sin):
    a = jnp.roll(x, -1, axis=1)
    b = jnp.roll(x, 1, axis=1)
    lane = jax.lax.broadcasted_iota(jnp.int32, x.shape, 1)
    rot = jnp.where(lane % 2 == 0, -a, b)
    return x * cos + rot * sin


def _attn_kernel(qn_ref, qp_ref, kn_ref, v_ref, kpe_ref, cos_ref, sin_ref,
                 o_ref):
    cos = cos_ref[...]
    sin = sin_ref[...]
    qpe = _rope(qp_ref[0], cos, sin)
    kpe = _rope(kpe_ref[...], cos, sin)
    s = _dot_t(qn_ref[0], kn_ref[0]) + _dot_t(qpe, kpe)
    s = s * (1.0 / math.sqrt(_QH))
    row = jax.lax.broadcasted_iota(jnp.int32, (_S, _S), 0)
    col = jax.lax.broadcasted_iota(jnp.int32, (_S, _S), 1)
    s = jnp.where(col > row, jnp.float32(-1e9), s)
    m = jnp.max(s, axis=1, keepdims=True)
    p = jnp.exp(s - m)
    p = p / jnp.sum(p, axis=1, keepdims=True)
    o_ref[0] = jax.lax.dot_general(p.astype(jnp.bfloat16),
                                   v_ref[0].astype(jnp.bfloat16),
                                   (((1,), (0,)), ((), ())),
                                   preferred_element_type=jnp.float32)


def _attention(qn, qp, kn, v, kpe, cos2, sin2):
    return pl.pallas_call(
        _attn_kernel,
        grid=(_H,),
        in_specs=[
            pl.BlockSpec((1, _S, _NOPE), lambda h: (h, 0, 0)),
            pl.BlockSpec((1, _S, _ROPE), lambda h: (h, 0, 0)),
            pl.BlockSpec((1, _S, _NOPE), lambda h: (h, 0, 0)),
            pl.BlockSpec((1, _S, _VH), lambda h: (h, 0, 0)),
            pl.BlockSpec((_S, _ROPE), lambda h: (0, 0)),
            pl.BlockSpec((_S, _ROPE), lambda h: (0, 0)),
            pl.BlockSpec((_S, _ROPE), lambda h: (0, 0)),
        ],
        out_specs=pl.BlockSpec((1, _S, _VH), lambda h: (h, 0, 0)),
        out_shape=jax.ShapeDtypeStruct((_H, _S, _VH), jnp.float32),
    )(qn, qp, kn, v, kpe, cos2, sin2)


def _oproj_kernel(ao_ref, ow_ref, x_ref, ln2w_ref, ln2b_ref, h1_ref, xf_ref):
    h = pl.program_id(0)
    part = _dot_t(ao_ref[0], ow_ref[...])

    @pl.when(h == 0)
    def _():
        h1_ref[...] = x_ref[...] + part

    @pl.when(h > 0)
    def _():
        h1_ref[...] += part

    @pl.when(h == _H - 1)
    def _():
        xf_ref[...] = _ln_body(h1_ref[...], ln2w_ref[0], ln2b_ref[0])


def _oproj(ao, o_w, x, ln2w, ln2b):
    return pl.pallas_call(
        _oproj_kernel,
        grid=(_H,),
        in_specs=[
            pl.BlockSpec((1, _S, _VH), lambda h: (h, 0, 0)),
            pl.BlockSpec((_D, _VH), lambda h: (0, h)),
            pl.BlockSpec((_S, _D), lambda h: (0, 0)),
            pl.BlockSpec((1, _D), lambda h: (0, 0)),
            pl.BlockSpec((1, _D), lambda h: (0, 0)),
        ],
        out_specs=[
            pl.BlockSpec((_S, _D), lambda h: (0, 0)),
            pl.BlockSpec((_S, _D), lambda h: (0, 0)),
        ],
        out_shape=[
            jax.ShapeDtypeStruct((_S, _D), jnp.float32),
            jax.ShapeDtypeStruct((_S, _D), jnp.float32),
        ],
    )(ao, o_w, x, ln2w.reshape(1, _D), ln2b.reshape(1, _D))


def _route_kernel(xf_ref, gw_ref, comb_ref):
    bm = xf_ref.shape[0]
    l = jax.lax.dot_general(xf_ref[...], gw_ref[...], (((1,), (1,)), ((), ())),
                            preferred_element_type=jnp.float32)
    ivec = jax.lax.broadcasted_iota(jnp.int32, (bm, _E), 1)
    r = jnp.zeros((bm, _E), jnp.float32)
    for j in range(_E):
        lj = jax.lax.slice_in_dim(l, j, j + 1, axis=1)
        beats = (lj > l) | ((lj == l) & (j < ivec))
        sg = (ivec // _GS) == (j // _GS)
        r = r + jnp.where(beats & sg, 1.0, 0.0)
    cand = r < _TKG
    pos = (ivec // _GS).astype(jnp.float32) * _TKG + r
    rr = jnp.zeros((bm, _E), jnp.float32)
    for j in range(_E):
        lj = jax.lax.slice_in_dim(l, j, j + 1, axis=1)
        pj = jax.lax.slice_in_dim(pos, j, j + 1, axis=1)
        cj = jax.lax.slice_in_dim(cand, j, j + 1, axis=1)
        beats2 = cj & ((lj > l) | ((lj == l) & (pj < pos)))
        rr = rr + jnp.where(beats2, 1.0, 0.0)
    sel = cand & (rr < _TOPK)
    w = jnp.where(sel, l, jnp.float32(0.0))
    comb_ref[...] = w / (jnp.sum(w, axis=1, keepdims=True) + 1e-20)


def _route(xf, gate_w, bm=512):
    return pl.pallas_call(
        _route_kernel,
        grid=(_S // bm,),
        in_specs=[
            pl.BlockSpec((bm, _D), lambda i: (i, 0)),
            pl.BlockSpec((_E, _D), lambda i: (0, 0)),
        ],
        out_specs=pl.BlockSpec((bm, _E), lambda i: (i, 0)),
        out_shape=jax.ShapeDtypeStruct((_S, _E), jnp.float32),
    )(xf, gate_w)


def _shared_kernel(xf_ref, h1_ref, gu_ref, dn_ref, o_ref):
    e = pl.program_id(0)
    h = _dot_t(xf_ref[...], gu_ref[0])
    g = h[:, :_INTER]
    u = h[:, _INTER:]
    act = (g / (1.0 + jnp.exp(-g))) * u
    y = _dot_t(act, dn_ref[0])

    @pl.when(e == 0)
    def _():
        o_ref[...] = h1_ref[...] + y

    @pl.when(e > 0)
    def _():
        o_ref[...] += y


def _shared(xf, h1, sh_gu, sh_dn):
    return pl.pallas_call(
        _shared_kernel,
        grid=(_NSH,),
        in_specs=[
            pl.BlockSpec((_S, _D), lambda e: (0, 0)),
            pl.BlockSpec((_S, _D), lambda e: (0, 0)),
            pl.BlockSpec((1, 2 * _INTER, _D), lambda e: (e, 0, 0)),
            pl.BlockSpec((1, _D, _INTER), lambda e: (e, 0, 0)),
        ],
        out_specs=pl.BlockSpec((_S, _D), lambda e: (0, 0)),
        out_shape=jax.ShapeDtypeStruct((_S, _D), jnp.float32),
    )(xf, h1, sh_gu, sh_dn)


def _routed_kernel(xf_ref, t1_ref, comb_ref, gu_ref, dn_ref, o_ref):
    e = pl.program_id(0)
    h = _dot_t(xf_ref[...], gu_ref[0])
    g = h[:, :_INTER]
    u = h[:, _INTER:]
    act = (g / (1.0 + jnp.exp(-g))) * u
    y = _dot_t(act, dn_ref[0])
    lane = jax.lax.broadcasted_iota(jnp.int32, (_S, _E), 1)
    c = jnp.sum(jnp.where(lane == e, comb_ref[...], 0.0), axis=1,
                keepdims=True)

    @pl.when(e == 0)
    def _():
        o_ref[...] = t1_ref[...] + c * y

    @pl.when(e > 0)
    def _():
        o_ref[...] += c * y


def _routed(xf, t1, comb, exp_gu, exp_dn):
    return pl.pallas_call(
        _routed_kernel,
        grid=(_E,),
        in_specs=[
            pl.BlockSpec((_S, _D), lambda e: (0, 0)),
            pl.BlockSpec((_S, _D), lambda e: (0, 0)),
            pl.BlockSpec((_S, _E), lambda e: (0, 0)),
            pl.BlockSpec((1, 2 * _INTER, _D), lambda e: (e, 0, 0)),
            pl.BlockSpec((1, _D, _INTER), lambda e: (e, 0, 0)),
        ],
        out_specs=pl.BlockSpec((_S, _D), lambda e: (0, 0)),
        out_shape=jax.ShapeDtypeStruct((_S, _D), jnp.float32),
    )(xf, t1, comb, exp_gu, exp_dn)


def _rope_tables():
    inv = 1.0 / (10000.0 ** (np.arange(0, _ROPE, 2, dtype=np.float64) / _ROPE))
    t = np.arange(_S, dtype=np.float64)
    f = np.outer(t, inv)
    cos2 = np.repeat(np.cos(f), 2, axis=1)
    sin2 = np.repeat(np.sin(f), 2, axis=1)
    return jnp.asarray(cos2, jnp.float32), jnp.asarray(sin2, jnp.float32)


def kernel(x, ln1_w, ln1_b, ln2_w, ln2_b, qkv_a_w, qa_ln_w, qa_ln_b, q_b_w,
           kv_a_w, kv_ln_w, kv_ln_b, kv_b_w, o_w, gate_w, exp_gu, exp_dn,
           sh_gu, sh_dn):
    x2 = x.reshape(_S, _D)
    cos2, sin2 = _rope_tables()

    q_lat = _ln_mm1(x2, ln1_w, ln1_b, qkv_a_w[:_QL])
    c_kv, k_pe = _ln_mm_kv(x2, ln1_w, ln1_b, kv_a_w)

    qn, qp = _ln_mm_heads(q_lat, qa_ln_w, qa_ln_b,
                          q_b_w.reshape(_H, _QH, _QL), _NOPE, _ROPE)
    kn, v3 = _ln_mm_heads(c_kv, kv_ln_w, kv_ln_b,
                          kv_b_w.reshape(_H, _NOPE + _VH, _KVL), _NOPE, _VH)

    ao = _attention(qn, qp, kn, v3, k_pe, cos2, sin2)
    h1, xf = _oproj(ao, o_w, x2, ln2_w, ln2_b)

    comb = _route(xf, gate_w)
    t1 = _shared(xf, h1, sh_gu, sh_dn)
    out = _routed(xf, t1, comb, exp_gu, exp_dn)

    return out.reshape(_B, _S, _D)
